# baseline (speedup 1.0000x reference)
import functools

import jax
import jax.numpy as jnp
from jax import lax
from jax.experimental import pallas as pl
from jax.experimental.pallas import tpu as pltpu
from jax.experimental.pallas import tpu_sc as plsc

OUT_PAD = 256
M = 8
S_TC = 0
R = 32
NW = 32


def _sc_reduce(point_flat, B, S, H):
    nrows = (S - S_TC) // 2
    chw = R * H
    nchunks = nrows // R
    assert nchunks % 2 == 0 and nrows % R == 0

    mesh = plsc.VectorSubcoreMesh(core_axis_name="c", subcore_axis_name="s")

    @functools.partial(
        pl.kernel,
        out_type=jax.ShapeDtypeStruct((2 * B * H,), jnp.float32),
        mesh=mesh,
        scratch_types=[
            pltpu.VMEM((chw,), jnp.float32),
            pltpu.VMEM((chw,), jnp.float32),
            pltpu.VMEM((H,), jnp.float32),
            pltpu.SemaphoreType.DMA,
            pltpu.SemaphoreType.DMA,
        ],
    )
    def sc_kernel(pe_hbm, out_hbm, buf0, buf1, acc, sem0, sem1):
        wid = lax.axis_index("s") * 2 + lax.axis_index("c")
        b = lax.rem(wid, B)
        half = wid // B
        base = (b * S + S_TC + half * nrows) * H

        @pl.loop(0, H, step=16)
        def _(g):
            acc[pl.ds(g, 16)] = jnp.full((16,), -jnp.inf, jnp.float32)

        def start(c, buf, sem):
            pltpu.async_copy(pe_hbm.at[pl.ds(base + c * chw, chw)], buf, sem)

        def wait(buf, sem):
            pltpu.make_async_copy(pe_hbm.at[pl.ds(base, chw)], buf, sem).wait()

        def consume(buf):
            @pl.loop(0, H, step=16)
            def _(g):
                m = buf[pl.ds(g, 16)]
                for r in range(1, R):
                    m = jnp.maximum(m, buf[pl.ds(r * H + g, 16)])
                acc[pl.ds(g, 16)] = jnp.maximum(acc[pl.ds(g, 16)], m)

        start(0, buf0, sem0)

        @pl.loop(0, nchunks, step=2)
        def _(c):
            start(c + 1, buf1, sem1)
            wait(buf0, sem0)
            consume(buf0)

            @pl.when(c + 2 < nchunks)
            def _():
                start(c + 2, buf0, sem0)

            wait(buf1, sem1)
            consume(buf1)

        pltpu.sync_copy(acc, out_hbm.at[pl.ds((half * B + b) * H, H)])

    return sc_kernel(point_flat)


def _tc_reduce_body(*refs):
    pe_refs = refs[:M]
    out_ref = refs[M]
    out_ref[...] = functools.reduce(
        jnp.maximum,
        [jnp.max(r[...], axis=0, keepdims=True) for r in pe_refs])


def _tc_reduce(point_embeds, B, S, H):
    ch = S_TC // M

    def pe_spec(i):
        return pl.BlockSpec((ch, H), lambda b, i=i: ((b * S) // ch + i, 0))

    return pl.pallas_call(
        _tc_reduce_body,
        grid=(B,),
        in_specs=[pe_spec(i) for i in range(M)],
        out_specs=pl.BlockSpec((1, H), lambda b: (b, 0)),
        out_shape=jax.ShapeDtypeStruct((B, H), jnp.float32),
    )(*([point_embeds] * M))


def _mlp_body(*refs):
    *part_refs, pos_ref, w1a_ref, w1p_ref, b1_ref, w2_ref, b2_ref, out_ref = refs
    x = functools.reduce(jnp.maximum, [r[...] for r in part_refs])
    h = jax.lax.dot_general(
        x, w1a_ref[...], (((1,), (0,)), ((), ())),
        preferred_element_type=jnp.float32)
    h += jax.lax.dot_general(
        pos_ref[...], w1p_ref[...], (((1,), (0,)), ((), ())),
        preferred_element_type=jnp.float32)
    h += b1_ref[...]
    h = jnp.where(h > 0, h, 0.02 * h)
    out = jax.lax.dot_general(
        h, w2_ref[...], (((1,), (0,)), ((), ())),
        preferred_element_type=jnp.float32)
    out_ref[...] = out + b2_ref[...]


def _mlp(partials, pos_condition, W1a, W1p, b1r, W2p, b2p, B, H):
    n = len(partials)
    whole = lambda shape: pl.BlockSpec(shape, lambda: tuple(0 for _ in shape))
    return pl.pallas_call(
        _mlp_body,
        grid=(),
        in_specs=[whole((B, H))] * n + [
            whole((B, 3)), whole((H, H)), whole((3, H)), whole((1, H)),
            whole((H, OUT_PAD)), whole((1, OUT_PAD)),
        ],
        out_specs=whole((B, OUT_PAD)),
        out_shape=jax.ShapeDtypeStruct((B, OUT_PAD), jnp.float32),
    )(*partials, pos_condition, W1a, W1p, b1r, W2p, b2p)


def kernel(point_embeds, npoints_in_batch, pos_condition, W1, b1, W2, b2):
    T, H = point_embeds.shape
    B = pos_condition.shape[0]
    S = T // B
    OUT = W2.shape[1]

    W1a = W1[:H]
    W1p = W1[H:]
    b1r = b1.reshape(1, H)
    W2p = jnp.pad(W2, ((0, 0), (0, OUT_PAD - OUT)))
    b2p = jnp.pad(b2, (0, OUT_PAD - OUT)).reshape(1, OUT_PAD)

    partials = []
    if S_TC < S:
        sc_flat = _sc_reduce(point_embeds.reshape(-1), B, S, H)
        sc_part = sc_flat.reshape(2, B, H)
        partials += [sc_part[0], sc_part[1]]
    if S_TC > 0:
        partials.append(_tc_reduce(point_embeds, B, S, H))

    out = _mlp(partials, pos_condition, W1a, W1p, b1r, W2p, b2p, B, H)

    action_embeds = out[:, :OUT]
    xr = action_embeds[..., : OUT - 1].reshape(-1, (OUT - 1) // 3, 3)
    xo = action_embeds[..., OUT - 1]
    return (xr, xo)
